# Initial kernel scaffold; baseline (speedup 1.0000x reference)
import jax
import jax.numpy as jnp
from jax.experimental import pallas as pl


def kernel(z_n, source_idx, target_idx, B, c, A, d):
    raise NotImplementedError("write your pallas kernel here")


# speedup vs baseline: 2.2155x; 2.2155x over previous
import jax
import jax.numpy as jnp
from jax import lax
from jax.experimental import pallas as pl
from jax.experimental.pallas import tpu as pltpu
from jax.experimental.pallas import tpu_sc as plsc

NUM_CHARTS = 100000
LATENT = 16
RANK = 16
BATCH = 16384

NUM_CORES = 2
NUM_SUBCORES = 16
NW = NUM_CORES * NUM_SUBCORES
PER_W = BATCH // NW
CH = 128
CHUNKS = PER_W // CH


def _body(z_hbm, si_hbm, ti_hbm, B_hbm, c_hbm, A_hbm, d_hbm, o_hbm,
          idx_s, idx_t, Bv, cv, Av, dv, zv, ov, sem):
    wid = lax.axis_index("s") * NUM_CORES + lax.axis_index("c")
    lane = lax.iota(jnp.int32, 16)

    @pl.loop(0, CHUNKS)
    def _(ch):
        base = wid * PER_W + ch * CH
        pltpu.sync_copy(si_hbm.at[pl.ds(base, CH)], idx_s)
        pltpu.sync_copy(ti_hbm.at[pl.ds(base, CH)], idx_t)
        pltpu.sync_copy(z_hbm.at[pl.ds(base, CH)], zv)
        cp1 = pltpu.async_copy(B_hbm.at[idx_s], Bv, sem)
        cp2 = pltpu.async_copy(c_hbm.at[idx_s], cv, sem)
        cp3 = pltpu.async_copy(A_hbm.at[idx_t], Av, sem)
        cp4 = pltpu.async_copy(d_hbm.at[idx_t], dv, sem)
        cp1.wait()
        cp2.wait()
        cp3.wait()
        cp4.wait()

        @pl.loop(0, CH)
        def _(i):
            z = zv[i]
            zg = cv[i]
            for r in range(RANK):
                s = jnp.sum(Bv[i, r] * z)
                zg = jnp.where(lane == r, zg + s, zg)
            o = dv[i]
            for r in range(LATENT):
                s = jnp.sum(Av[i, r] * zg)
                o = jnp.where(lane == r, o + s, o)
            ov[i] = o

        pltpu.sync_copy(ov, o_hbm.at[pl.ds(base, CH)])


def kernel(z_n, source_idx, target_idx, B, c, A, d):
    mesh = plsc.VectorSubcoreMesh(core_axis_name="c", subcore_axis_name="s")
    k = pl.kernel(
        _body,
        out_type=jax.ShapeDtypeStruct((BATCH, LATENT), jnp.float32),
        mesh=mesh,
        compiler_params=pltpu.CompilerParams(
            needs_layout_passes=False, use_tc_tiling_on_sc=False),
        scratch_types=[
            pltpu.VMEM((CH,), jnp.int32),
            pltpu.VMEM((CH,), jnp.int32),
            pltpu.VMEM((CH, RANK, LATENT), jnp.float32),
            pltpu.VMEM((CH, RANK), jnp.float32),
            pltpu.VMEM((CH, LATENT, RANK), jnp.float32),
            pltpu.VMEM((CH, LATENT), jnp.float32),
            pltpu.VMEM((CH, LATENT), jnp.float32),
            pltpu.VMEM((CH, LATENT), jnp.float32),
            pltpu.SemaphoreType.DMA,
        ],
    )
    return k(z_n, source_idx.astype(jnp.int32), target_idx.astype(jnp.int32),
             B, c, A, d)


# speedup vs baseline: 7.0750x; 3.1935x over previous
import jax
import jax.numpy as jnp
from jax import lax
from jax.experimental import pallas as pl
from jax.experimental.pallas import tpu as pltpu
from jax.experimental.pallas import tpu_sc as plsc

NUM_CHARTS = 100000
LATENT = 16
RANK = 16
BATCH = 16384

NUM_CORES = 2
NUM_SUBCORES = 16
NW = NUM_CORES * NUM_SUBCORES
PER_W = BATCH // NW
CH = 128
CHUNKS = PER_W // CH


def _body(z_hbm, si_hbm, ti_hbm, B_hbm, c_hbm, A_hbm, d_hbm, o_hbm,
          idx_s, idx_t, Bv, cv, Av, dv, zv, ov, sem):
    wid = lax.axis_index("s") * NUM_CORES + lax.axis_index("c")
    lane = lax.iota(jnp.int32, 16)

    @pl.loop(0, CHUNKS)
    def _(ch):
        base = wid * PER_W + ch * CH
        pltpu.sync_copy(si_hbm.at[pl.ds(base, CH)], idx_s)
        pltpu.sync_copy(ti_hbm.at[pl.ds(base, CH)], idx_t)
        pltpu.sync_copy(z_hbm.at[pl.ds(base, CH)], zv)
        cp1 = pltpu.async_copy(B_hbm.at[idx_s], Bv, sem)
        cp2 = pltpu.async_copy(c_hbm.at[idx_s], cv, sem)
        cp3 = pltpu.async_copy(A_hbm.at[idx_t], Av, sem)
        cp4 = pltpu.async_copy(d_hbm.at[idx_t], dv, sem)
        cp1.wait()
        cp2.wait()
        cp3.wait()
        cp4.wait()

        @pl.loop(0, CH)
        def _(i):
            z = zv[i]
            zg = cv[i]
            for r in range(RANK):
                s = jnp.sum(Bv[i, pl.ds(r * LATENT, LATENT)] * z)
                zg = jnp.where(lane == r, zg + s, zg)
            o = dv[i]
            for r in range(LATENT):
                s = jnp.sum(Av[i, pl.ds(r * RANK, RANK)] * zg)
                o = jnp.where(lane == r, o + s, o)
            ov[i] = o

        pltpu.sync_copy(ov, o_hbm.at[pl.ds(base, CH)])


def kernel(z_n, source_idx, target_idx, B, c, A, d):
    mesh = plsc.VectorSubcoreMesh(core_axis_name="c", subcore_axis_name="s")
    k = pl.kernel(
        _body,
        out_type=jax.ShapeDtypeStruct((BATCH, LATENT), jnp.float32),
        mesh=mesh,
        compiler_params=pltpu.CompilerParams(
            needs_layout_passes=False, use_tc_tiling_on_sc=False),
        scratch_types=[
            pltpu.VMEM((CH,), jnp.int32),
            pltpu.VMEM((CH,), jnp.int32),
            pltpu.VMEM((CH, RANK * LATENT), jnp.float32),
            pltpu.VMEM((CH, RANK), jnp.float32),
            pltpu.VMEM((CH, LATENT * RANK), jnp.float32),
            pltpu.VMEM((CH, LATENT), jnp.float32),
            pltpu.VMEM((CH, LATENT), jnp.float32),
            pltpu.VMEM((CH, LATENT), jnp.float32),
            pltpu.SemaphoreType.DMA,
        ],
    )
    return k(z_n, source_idx.astype(jnp.int32), target_idx.astype(jnp.int32),
             B.reshape(NUM_CHARTS, RANK * LATENT), c,
             A.reshape(NUM_CHARTS, LATENT * RANK), d)
